# baseline (speedup 1.0000x reference)
import functools

import jax
import jax.numpy as jnp
from jax import lax
from jax.experimental import pallas as pl
from jax.experimental.pallas import tpu as pltpu
from jax.experimental.pallas import tpu_sc as plsc

NC = 2
NS = 16
L = 16


def _p1_body(x_ref, t_ref, se_ref, nr_ref):
    x = x_ref[...]
    t = t_ref[...]
    d = x - t
    ones8 = jnp.ones((8, x.shape[1]), jnp.float32)
    dn = (((1,), (1,)), ((), ()))
    se8 = lax.dot_general(ones8, d * d, dn, preferred_element_type=jnp.float32)
    nr8 = lax.dot_general(ones8, t * t, dn, preferred_element_type=jnp.float32)
    se_ref[...] = se8[0, :]
    nr_ref[...] = jnp.sqrt(nr8[0, :])


def _pass1(x, t, blk=8192):
    n, d = x.shape
    return pl.pallas_call(
        _p1_body,
        grid=(n // blk,),
        in_specs=[pl.BlockSpec((blk, d), lambda i: (i, 0))] * 2,
        out_specs=[pl.BlockSpec((blk,), lambda i: (i,))] * 2,
        out_shape=[jax.ShapeDtypeStruct((n,), jnp.float32)] * 2,
    )(x, t)


_R1_BINS, _R2_BINS, _R3_BINS = 2048, 2048, 512
_U = 8


def _make_quantile_kernel(n, nq):
    per_tile = n // NS
    assert per_tile * NS == n and per_tile % L == 0
    mesh = plsc.VectorSubcoreMesh(
        core_axis_name="c", subcore_axis_name="s", num_cores=NC, num_subcores=NS
    )

    @functools.partial(
        pl.kernel,
        out_type=jax.ShapeDtypeStruct((NC, L), jnp.float32),
        mesh=mesh,
        compiler_params=pltpu.CompilerParams(needs_layout_passes=False),
        scratch_types=[
            pltpu.VMEM((per_tile,), jnp.float32),
            pltpu.VMEM((1, _U * 4096), jnp.int32),
            pltpu.VMEM((1, 4096), jnp.int32),
            pltpu.VMEM((L,), jnp.float32),
            pltpu.VMEM((1,), jnp.int32),
            pltpu.VMEM((L,), jnp.float32),
            pltpu.VMEM_SHARED((1, 4096), jnp.int32),
        ],
    )
    def qkernel(norms_hbm, q_hbm, zero1_hbm, out_hbm,
                nrm_v, hist_v, mhist_v, q_v, idx0_v, res_v, shared):
        cid = lax.axis_index("c")
        sid = lax.axis_index("s")

        pltpu.sync_copy(norms_hbm.at[pl.ds(sid * per_tile, per_tile)], nrm_v)
        pltpu.sync_copy(q_hbm, q_v.at[pl.ds(0, nq)])
        pltpu.sync_copy(zero1_hbm, idx0_v)

        lane = lax.iota(jnp.int32, L)
        zeros16 = jnp.zeros((L,), jnp.int32)
        ones16 = jnp.ones((L,), jnp.int32)
        idx15 = jnp.full((L,), L - 1, jnp.int32)

        qv = q_v[...]
        q0 = jnp.max(jnp.where(lane == 0, qv, 0.0))
        q2 = jnp.max(jnp.where(lane == nq - 1, qv, 0.0))
        qc = jnp.where(cid == 0, q0, q2)
        idxf = jnp.full((L,), qc) * jnp.float32(n - 1)
        ka_vec = idxf.astype(jnp.int32)
        frac_vec = idxf - ka_vec.astype(jnp.float32)
        kb_vec = jnp.minimum(ka_vec + 1, n - 1)

        def begin_round(nsub, nwords):
            @plsc.parallel_loop(0, nwords // L, unroll=4)
            def _(i):
                for u in range(nsub):
                    hist_v[0, pl.ds(u * 4096 + i * L, L)] = zeros16

            @plsc.parallel_loop(0, 4096 // L, unroll=8)
            def _(i):
                mhist_v[0, pl.ds(i * L, L)] = zeros16

            @pl.when(sid == 0)
            def _():
                pltpu.sync_copy(mhist_v, shared)
            plsc.subcore_barrier()

        def merge_hist(nsub, nwords):
            @plsc.parallel_loop(0, nwords // L, unroll=4)
            def _(i):
                acc = hist_v[0, pl.ds(i * L, L)]
                for u in range(1, nsub):
                    acc = acc + hist_v[0, pl.ds(u * 4096 + i * L, L)]
                mhist_v[0, pl.ds(i * L, L)] = acc
            pltpu.sync_copy(mhist_v, shared.at[idx0_v], add=True)
            plsc.subcore_barrier()
            pltpu.sync_copy(shared, mhist_v)
            plsc.subcore_barrier()

        def _splat_last(cum):
            return lax.gather(
                cum, idx15[:, None],
                lax.GatherDimensionNumbers(
                    offset_dims=(), collapsed_slice_dims=(0,),
                    start_index_map=(0,)),
                (1,), mode=lax.GatherScatterMode.PROMISE_IN_BOUNDS)

        def scan_multi(specs, nchunks):
            def sbody(i, carry):
                ci = 0
                new_carry = []
                for base, _, kvecs in specs:
                    tot = carry[ci]
                    ci += 1
                    h = mhist_v[0, pl.ds(base + i * L, L)]
                    cum = plsc.cumsum(h) + tot
                    new_carry.append(_splat_last(cum))
                    for kv in kvecs:
                        mask = cum <= kv
                        new_carry.append(carry[ci] + jnp.where(mask, 1, 0))
                        new_carry.append(jnp.maximum(carry[ci + 1],
                                                     jnp.where(mask, cum, 0)))
                        ci += 2
                return tuple(new_carry)

            init = []
            for _, tot0, kvecs in specs:
                init.append(tot0)
                init.extend([zeros16] * (2 * len(kvecs)))
            out = lax.fori_loop(0, nchunks, sbody, tuple(init), unroll=2)
            res = []
            ci = 0
            for _, _, kvecs in specs:
                ci += 1
                cur = []
                for _ in kvecs:
                    cur.append((jnp.sum(out[ci]), jnp.max(out[ci + 1])))
                    ci += 2
                res.append(cur)
            return res

        begin_round(_U, _R1_BINS)

        @plsc.parallel_loop(0, per_tile // L, unroll=_U)
        def _(i):
            v = nrm_v[pl.ds(i * L, L)]
            bits = plsc.bitcast(v, jnp.int32)
            hi = bits >> 20
            u = (i % _U) * 4096
            plsc.addupdate_scatter(hist_v, [zeros16, u + hi], ones16)
        merge_hist(_U, _R1_BINS)

        half = _R1_BINS // 2

        def pbody(i, acc):
            return acc + mhist_v[0, pl.ds(i * L, L)]
        accv = lax.fori_loop(0, half // L, pbody, zeros16, unroll=8)
        htvec = jnp.full((L,), jnp.sum(accv))

        r1 = scan_multi([(0, zeros16, [ka_vec, kb_vec]),
                         (half, htvec, [ka_vec, kb_vec])], half // L)
        b1a = r1[0][0][0] + r1[1][0][0]
        c1a = jnp.maximum(r1[0][0][1], r1[1][0][1])
        b1b = r1[0][1][0] + r1[1][1][0]
        c1b = jnp.maximum(r1[0][1][1], r1[1][1][1])
        ka = jnp.max(ka_vec)
        kb = jnp.max(kb_vec)
        r2a_vec = jnp.full((L,), ka - c1a)
        r2b_vec = jnp.full((L,), kb - c1b)

        begin_round(_U, 2 * _R2_BINS)

        @plsc.parallel_loop(0, per_tile // L, unroll=_U)
        def _(i):
            v = nrm_v[pl.ds(i * L, L)]
            bits = plsc.bitcast(v, jnp.int32)
            hi = bits >> 20
            mid = (bits >> 9) & 0x7FF
            u = (i % _U) * 4096
            plsc.addupdate_scatter(hist_v, [zeros16, u + mid],
                                   ones16, mask=hi == b1a)
            plsc.addupdate_scatter(hist_v, [zeros16, u + _R2_BINS + mid],
                                   ones16, mask=hi == b1b)
        merge_hist(_U, 2 * _R2_BINS)

        r2 = scan_multi([(0, zeros16, [r2a_vec]),
                         (_R2_BINS, zeros16, [r2b_vec])], _R2_BINS // L)
        b2a, c2a = r2[0][0]
        b2b, c2b = r2[1][0]
        r3a_vec = r2a_vec - c2a
        r3b_vec = r2b_vec - c2b

        begin_round(_U, 2 * _R2_BINS)

        @plsc.parallel_loop(0, per_tile // L, unroll=_U)
        def _(i):
            v = nrm_v[pl.ds(i * L, L)]
            bits = plsc.bitcast(v, jnp.int32)
            hi = bits >> 20
            mid = (bits >> 9) & 0x7FF
            lo = bits & 0x1FF
            u = (i % _U) * 4096
            plsc.addupdate_scatter(hist_v, [zeros16, u + lo], ones16,
                                   mask=(hi == b1a) & (mid == b2a))
            plsc.addupdate_scatter(hist_v, [zeros16, u + _R2_BINS + lo],
                                   ones16, mask=(hi == b1b) & (mid == b2b))
        merge_hist(_U, 2 * _R2_BINS)

        r3 = scan_multi([(0, zeros16, [r3a_vec]),
                         (_R2_BINS, zeros16, [r3b_vec])], _R3_BINS // L)
        b3a = r3[0][0][0]
        b3b = r3[1][0][0]

        bits_a = (b1a << 20) | (b2a << 9) | b3a
        bits_b = (b1b << 20) | (b2b << 9) | b3b
        va = plsc.bitcast(jnp.full((L,), bits_a, jnp.int32), jnp.float32)
        vb = plsc.bitcast(jnp.full((L,), bits_b, jnp.int32), jnp.float32)
        res_v[...] = va + frac_vec * (vb - va)

        @pl.when(sid == 0)
        def _():
            pltpu.sync_copy(res_v, out_hbm.at[cid])

    return qkernel


def _make_p3_body(scale):
    def _p3_body(nr_ref, se_ref, qv_ref, pq_ref, w_ref, out_ref):
        i = pl.program_id(0)
        n = nr_ref[...]
        se = se_ref[...]
        q_lo = qv_ref[0, 0]
        q_hi = qv_ref[1, 0]
        tw = jnp.where(n < q_lo, w_ref[0], 0.0)
        tw = jnp.where((n >= pq_ref[1]) & (n < pq_ref[2]), w_ref[1], tw)
        tw = jnp.where(n > q_hi, w_ref[2], tw)
        part = jnp.sum(tw * se).reshape(1, 1)

        @pl.when(i == 0)
        def _():
            out_ref[...] = jnp.zeros((1, 1), jnp.float32)

        out_ref[...] += part

        @pl.when(i == pl.num_programs(0) - 1)
        def _():
            out_ref[...] = out_ref[...] * scale
    return _p3_body


def _pass3(norms, sqerr, qv, pq, w, total, blk=65536):
    n = norms.shape[0]
    smem = pl.BlockSpec(memory_space=pltpu.SMEM)
    return pl.pallas_call(
        _make_p3_body(1.0 / total),
        grid=(n // blk,),
        in_specs=[
            pl.BlockSpec((blk,), lambda i: (i,)),
            pl.BlockSpec((blk,), lambda i: (i,)),
            smem, smem, smem,
        ],
        out_specs=pl.BlockSpec((1, 1), lambda i: (0, 0)),
        out_shape=jax.ShapeDtypeStruct((1, 1), jnp.float32),
    )(norms, sqerr, qv, pq, w)


def kernel(input, target, quantiles, weights):
    n, d = target.shape
    sqerr, norms = _pass1(input, target)

    zero1 = jnp.zeros((1,), jnp.int32)
    qv = _make_quantile_kernel(n, int(quantiles.shape[0]))(
        norms, quantiles.astype(jnp.float32), zero1)
    loss = _pass3(norms, sqerr, qv, quantiles.astype(jnp.float32),
                  weights.astype(jnp.float32), float(n) * float(d))
    return loss.reshape(())


# speedup vs baseline: 1.0376x; 1.0376x over previous
import functools

import jax
import jax.numpy as jnp
from jax import lax
from jax.experimental import pallas as pl
from jax.experimental.pallas import tpu as pltpu
from jax.experimental.pallas import tpu_sc as plsc

NC = 2
NS = 16
L = 16


def _p1_body(x_ref, t_ref, se_ref, nr_ref):
    x = x_ref[...]
    t = t_ref[...]
    d = x - t
    ones8 = jnp.ones((8, x.shape[1]), jnp.float32)
    dn = (((1,), (1,)), ((), ()))
    se8 = lax.dot_general(ones8, d * d, dn, preferred_element_type=jnp.float32)
    nr8 = lax.dot_general(ones8, t * t, dn, preferred_element_type=jnp.float32)
    se_ref[...] = se8[0, :]
    nr_ref[...] = jnp.sqrt(nr8[0, :])


def _pass1(x, t, blk=8192):
    n, d = x.shape
    return pl.pallas_call(
        _p1_body,
        grid=(n // blk,),
        in_specs=[pl.BlockSpec((blk, d), lambda i: (i, 0))] * 2,
        out_specs=[pl.BlockSpec((blk,), lambda i: (i,))] * 2,
        out_shape=[jax.ShapeDtypeStruct((n,), jnp.float32)] * 2,
    )(x, t)


_R1_BINS, _R2_BINS, _R3_BINS = 2048, 2048, 512
_U = 4


def _make_quantile_kernel(n, nq):
    per_tile = n // NS
    assert per_tile * NS == n and per_tile % L == 0
    mesh = plsc.VectorSubcoreMesh(
        core_axis_name="c", subcore_axis_name="s", num_cores=NC, num_subcores=NS
    )

    @functools.partial(
        pl.kernel,
        out_type=jax.ShapeDtypeStruct((NC, L), jnp.float32),
        mesh=mesh,
        compiler_params=pltpu.CompilerParams(needs_layout_passes=False),
        scratch_types=[
            pltpu.VMEM((per_tile,), jnp.float32),
            pltpu.VMEM((1, _U * 4096), jnp.int32),
            pltpu.VMEM((1, 4096), jnp.int32),
            pltpu.VMEM((L,), jnp.float32),
            pltpu.VMEM((1,), jnp.int32),
            pltpu.VMEM((L,), jnp.float32),
            pltpu.VMEM_SHARED((1, 4096), jnp.int32),
        ],
    )
    def qkernel(norms_hbm, q_hbm, zero1_hbm, out_hbm,
                nrm_v, hist_v, mhist_v, q_v, idx0_v, res_v, shared):
        cid = lax.axis_index("c")
        sid = lax.axis_index("s")

        pltpu.sync_copy(norms_hbm.at[pl.ds(sid * per_tile, per_tile)], nrm_v)
        pltpu.sync_copy(q_hbm, q_v.at[pl.ds(0, nq)])
        pltpu.sync_copy(zero1_hbm, idx0_v)

        lane = lax.iota(jnp.int32, L)
        zeros16 = jnp.zeros((L,), jnp.int32)
        ones16 = jnp.ones((L,), jnp.int32)
        idx15 = jnp.full((L,), L - 1, jnp.int32)

        qv = q_v[...]
        q0 = jnp.max(jnp.where(lane == 0, qv, 0.0))
        q2 = jnp.max(jnp.where(lane == nq - 1, qv, 0.0))
        qc = jnp.where(cid == 0, q0, q2)
        idxf = jnp.full((L,), qc) * jnp.float32(n - 1)
        ka_vec = idxf.astype(jnp.int32)
        frac_vec = idxf - ka_vec.astype(jnp.float32)
        kb_vec = jnp.minimum(ka_vec + 1, n - 1)

        def begin_round(nsub, nwords):
            @plsc.parallel_loop(0, nwords // L, unroll=4)
            def _(i):
                for u in range(nsub):
                    hist_v[0, pl.ds(u * 4096 + i * L, L)] = zeros16

            @plsc.parallel_loop(0, 4096 // L, unroll=8)
            def _(i):
                mhist_v[0, pl.ds(i * L, L)] = zeros16

            @pl.when(sid == 0)
            def _():
                pltpu.sync_copy(mhist_v, shared)
            plsc.subcore_barrier()

        def merge_hist(nsub, nwords):
            @plsc.parallel_loop(0, nwords // L, unroll=4)
            def _(i):
                acc = hist_v[0, pl.ds(i * L, L)]
                for u in range(1, nsub):
                    acc = acc + hist_v[0, pl.ds(u * 4096 + i * L, L)]
                mhist_v[0, pl.ds(i * L, L)] = acc
            pltpu.sync_copy(mhist_v, shared.at[idx0_v], add=True)
            plsc.subcore_barrier()
            pltpu.sync_copy(shared, mhist_v)
            plsc.subcore_barrier()

        def _splat_last(cum):
            return lax.gather(
                cum, idx15[:, None],
                lax.GatherDimensionNumbers(
                    offset_dims=(), collapsed_slice_dims=(0,),
                    start_index_map=(0,)),
                (1,), mode=lax.GatherScatterMode.PROMISE_IN_BOUNDS)

        def scan_multi(specs, nchunks):
            def sbody(i, carry):
                ci = 0
                new_carry = []
                for base, _, kvecs in specs:
                    tot = carry[ci]
                    ci += 1
                    h = mhist_v[0, pl.ds(base + i * L, L)]
                    cum = plsc.cumsum(h) + tot
                    new_carry.append(_splat_last(cum))
                    for kv in kvecs:
                        mask = cum <= kv
                        new_carry.append(carry[ci] + jnp.where(mask, 1, 0))
                        new_carry.append(jnp.maximum(carry[ci + 1],
                                                     jnp.where(mask, cum, 0)))
                        ci += 2
                return tuple(new_carry)

            init = []
            for _, tot0, kvecs in specs:
                init.append(tot0)
                init.extend([zeros16] * (2 * len(kvecs)))
            out = lax.fori_loop(0, nchunks, sbody, tuple(init), unroll=2)
            res = []
            ci = 0
            for _, _, kvecs in specs:
                ci += 1
                cur = []
                for _ in kvecs:
                    cur.append((jnp.sum(out[ci]), jnp.max(out[ci + 1])))
                    ci += 2
                res.append(cur)
            return res

        begin_round(_U, _R1_BINS)

        @plsc.parallel_loop(0, per_tile // L, unroll=_U)
        def _(i):
            v = nrm_v[pl.ds(i * L, L)]
            bits = plsc.bitcast(v, jnp.int32)
            hi = bits >> 20
            u = (i % _U) * 4096
            plsc.addupdate_scatter(hist_v, [zeros16, u + hi], ones16)
        merge_hist(_U, _R1_BINS)

        half = _R1_BINS // 2

        def pbody(i, acc):
            return acc + mhist_v[0, pl.ds(i * L, L)]
        accv = lax.fori_loop(0, half // L, pbody, zeros16, unroll=8)
        htvec = jnp.full((L,), jnp.sum(accv))

        r1 = scan_multi([(0, zeros16, [ka_vec, kb_vec]),
                         (half, htvec, [ka_vec, kb_vec])], half // L)
        b1a = r1[0][0][0] + r1[1][0][0]
        c1a = jnp.maximum(r1[0][0][1], r1[1][0][1])
        b1b = r1[0][1][0] + r1[1][1][0]
        c1b = jnp.maximum(r1[0][1][1], r1[1][1][1])
        ka = jnp.max(ka_vec)
        kb = jnp.max(kb_vec)
        r2a_vec = jnp.full((L,), ka - c1a)
        r2b_vec = jnp.full((L,), kb - c1b)

        begin_round(_U, 2 * _R2_BINS)

        @plsc.parallel_loop(0, per_tile // L, unroll=_U)
        def _(i):
            v = nrm_v[pl.ds(i * L, L)]
            bits = plsc.bitcast(v, jnp.int32)
            hi = bits >> 20
            mid = (bits >> 9) & 0x7FF
            u = (i % _U) * 4096
            plsc.addupdate_scatter(hist_v, [zeros16, u + mid],
                                   ones16, mask=hi == b1a)
            plsc.addupdate_scatter(hist_v, [zeros16, u + _R2_BINS + mid],
                                   ones16, mask=hi == b1b)
        merge_hist(_U, 2 * _R2_BINS)

        r2 = scan_multi([(0, zeros16, [r2a_vec]),
                         (_R2_BINS, zeros16, [r2b_vec])], _R2_BINS // L)
        b2a, c2a = r2[0][0]
        b2b, c2b = r2[1][0]
        r3a_vec = r2a_vec - c2a
        r3b_vec = r2b_vec - c2b

        begin_round(_U, 2 * _R2_BINS)

        @plsc.parallel_loop(0, per_tile // L, unroll=_U)
        def _(i):
            v = nrm_v[pl.ds(i * L, L)]
            bits = plsc.bitcast(v, jnp.int32)
            hi = bits >> 20
            mid = (bits >> 9) & 0x7FF
            lo = bits & 0x1FF
            u = (i % _U) * 4096
            plsc.addupdate_scatter(hist_v, [zeros16, u + lo], ones16,
                                   mask=(hi == b1a) & (mid == b2a))
            plsc.addupdate_scatter(hist_v, [zeros16, u + _R2_BINS + lo],
                                   ones16, mask=(hi == b1b) & (mid == b2b))
        merge_hist(_U, 2 * _R2_BINS)

        r3 = scan_multi([(0, zeros16, [r3a_vec]),
                         (_R2_BINS, zeros16, [r3b_vec])], _R3_BINS // L)
        b3a = r3[0][0][0]
        b3b = r3[1][0][0]

        bits_a = (b1a << 20) | (b2a << 9) | b3a
        bits_b = (b1b << 20) | (b2b << 9) | b3b
        va = plsc.bitcast(jnp.full((L,), bits_a, jnp.int32), jnp.float32)
        vb = plsc.bitcast(jnp.full((L,), bits_b, jnp.int32), jnp.float32)
        res_v[...] = va + frac_vec * (vb - va)

        @pl.when(sid == 0)
        def _():
            pltpu.sync_copy(res_v, out_hbm.at[cid])

    return qkernel


def _make_p3_body(scale):
    def _p3_body(nr_ref, se_ref, qv_ref, pq_ref, w_ref, out_ref):
        i = pl.program_id(0)
        n = nr_ref[...]
        se = se_ref[...]
        q_lo = qv_ref[0, 0]
        q_hi = qv_ref[1, 0]
        tw = jnp.where(n < q_lo, w_ref[0], 0.0)
        tw = jnp.where((n >= pq_ref[1]) & (n < pq_ref[2]), w_ref[1], tw)
        tw = jnp.where(n > q_hi, w_ref[2], tw)
        part = jnp.sum(tw * se).reshape(1, 1)

        @pl.when(i == 0)
        def _():
            out_ref[...] = jnp.zeros((1, 1), jnp.float32)

        out_ref[...] += part

        @pl.when(i == pl.num_programs(0) - 1)
        def _():
            out_ref[...] = out_ref[...] * scale
    return _p3_body


def _pass3(norms, sqerr, qv, pq, w, total, blk=65536):
    n = norms.shape[0]
    smem = pl.BlockSpec(memory_space=pltpu.SMEM)
    return pl.pallas_call(
        _make_p3_body(1.0 / total),
        grid=(n // blk,),
        in_specs=[
            pl.BlockSpec((blk,), lambda i: (i,)),
            pl.BlockSpec((blk,), lambda i: (i,)),
            smem, smem, smem,
        ],
        out_specs=pl.BlockSpec((1, 1), lambda i: (0, 0)),
        out_shape=jax.ShapeDtypeStruct((1, 1), jnp.float32),
    )(norms, sqerr, qv, pq, w)


def kernel(input, target, quantiles, weights):
    n, d = target.shape
    sqerr, norms = _pass1(input, target)

    zero1 = jnp.zeros((1,), jnp.int32)
    qv = _make_quantile_kernel(n, int(quantiles.shape[0]))(
        norms, quantiles.astype(jnp.float32), zero1)
    loss = _pass3(norms, sqerr, qv, quantiles.astype(jnp.float32),
                  weights.astype(jnp.float32), float(n) * float(d))
    return loss.reshape(())


# speedup vs baseline: 1.0447x; 1.0069x over previous
import functools

import jax
import jax.numpy as jnp
from jax import lax
from jax.experimental import pallas as pl
from jax.experimental.pallas import tpu as pltpu
from jax.experimental.pallas import tpu_sc as plsc

NC = 2
NS = 16
L = 16


def _p1_body(x_ref, t_ref, se_ref, nr_ref):
    x = x_ref[...]
    t = t_ref[...]
    d = x - t
    ones8 = jnp.ones((8, x.shape[1]), jnp.float32)
    dn = (((1,), (1,)), ((), ()))
    se8 = lax.dot_general(ones8, d * d, dn, preferred_element_type=jnp.float32)
    nr8 = lax.dot_general(ones8, t * t, dn, preferred_element_type=jnp.float32)
    se_ref[...] = se8[0, :]
    nr_ref[...] = jnp.sqrt(nr8[0, :])


def _pass1(x, t, blk=8192):
    n, d = x.shape
    return pl.pallas_call(
        _p1_body,
        grid=(n // blk,),
        in_specs=[pl.BlockSpec((blk, d), lambda i: (i, 0))] * 2,
        out_specs=[pl.BlockSpec((blk,), lambda i: (i,))] * 2,
        out_shape=[jax.ShapeDtypeStruct((n,), jnp.float32)] * 2,
    )(x, t)


_R1_BINS, _R2_BINS, _R3_BINS = 2048, 2048, 512
_U = 2


def _make_quantile_kernel(n, nq):
    per_tile = n // NS
    assert per_tile * NS == n and per_tile % L == 0
    mesh = plsc.VectorSubcoreMesh(
        core_axis_name="c", subcore_axis_name="s", num_cores=NC, num_subcores=NS
    )

    @functools.partial(
        pl.kernel,
        out_type=jax.ShapeDtypeStruct((NC, L), jnp.float32),
        mesh=mesh,
        compiler_params=pltpu.CompilerParams(needs_layout_passes=False),
        scratch_types=[
            pltpu.VMEM((per_tile,), jnp.float32),
            pltpu.VMEM((1, _U * 4096), jnp.int32),
            pltpu.VMEM((1, 4096), jnp.int32),
            pltpu.VMEM((L,), jnp.float32),
            pltpu.VMEM((1,), jnp.int32),
            pltpu.VMEM((L,), jnp.float32),
            pltpu.VMEM_SHARED((1, 4096), jnp.int32),
        ],
    )
    def qkernel(norms_hbm, q_hbm, zero1_hbm, out_hbm,
                nrm_v, hist_v, mhist_v, q_v, idx0_v, res_v, shared):
        cid = lax.axis_index("c")
        sid = lax.axis_index("s")

        pltpu.sync_copy(norms_hbm.at[pl.ds(sid * per_tile, per_tile)], nrm_v)
        pltpu.sync_copy(q_hbm, q_v.at[pl.ds(0, nq)])
        pltpu.sync_copy(zero1_hbm, idx0_v)

        lane = lax.iota(jnp.int32, L)
        zeros16 = jnp.zeros((L,), jnp.int32)
        ones16 = jnp.ones((L,), jnp.int32)
        idx15 = jnp.full((L,), L - 1, jnp.int32)

        qv = q_v[...]
        q0 = jnp.max(jnp.where(lane == 0, qv, 0.0))
        q2 = jnp.max(jnp.where(lane == nq - 1, qv, 0.0))
        qc = jnp.where(cid == 0, q0, q2)
        idxf = jnp.full((L,), qc) * jnp.float32(n - 1)
        ka_vec = idxf.astype(jnp.int32)
        frac_vec = idxf - ka_vec.astype(jnp.float32)
        kb_vec = jnp.minimum(ka_vec + 1, n - 1)

        def begin_round(nsub, nwords):
            @plsc.parallel_loop(0, nwords // L, unroll=4)
            def _(i):
                for u in range(nsub):
                    hist_v[0, pl.ds(u * 4096 + i * L, L)] = zeros16

            @plsc.parallel_loop(0, 4096 // L, unroll=8)
            def _(i):
                mhist_v[0, pl.ds(i * L, L)] = zeros16

            @pl.when(sid == 0)
            def _():
                pltpu.sync_copy(mhist_v, shared)
            plsc.subcore_barrier()

        def merge_hist(nsub, nwords):
            @plsc.parallel_loop(0, nwords // L, unroll=4)
            def _(i):
                acc = hist_v[0, pl.ds(i * L, L)]
                for u in range(1, nsub):
                    acc = acc + hist_v[0, pl.ds(u * 4096 + i * L, L)]
                mhist_v[0, pl.ds(i * L, L)] = acc
            pltpu.sync_copy(mhist_v, shared.at[idx0_v], add=True)
            plsc.subcore_barrier()
            pltpu.sync_copy(shared, mhist_v)
            plsc.subcore_barrier()

        def _splat_last(cum):
            return lax.gather(
                cum, idx15[:, None],
                lax.GatherDimensionNumbers(
                    offset_dims=(), collapsed_slice_dims=(0,),
                    start_index_map=(0,)),
                (1,), mode=lax.GatherScatterMode.PROMISE_IN_BOUNDS)

        def scan_multi(specs, nchunks):
            def sbody(i, carry):
                ci = 0
                new_carry = []
                for base, _, kvecs in specs:
                    tot = carry[ci]
                    ci += 1
                    h = mhist_v[0, pl.ds(base + i * L, L)]
                    cum = plsc.cumsum(h) + tot
                    new_carry.append(_splat_last(cum))
                    for kv in kvecs:
                        mask = cum <= kv
                        new_carry.append(carry[ci] + jnp.where(mask, 1, 0))
                        new_carry.append(jnp.maximum(carry[ci + 1],
                                                     jnp.where(mask, cum, 0)))
                        ci += 2
                return tuple(new_carry)

            init = []
            for _, tot0, kvecs in specs:
                init.append(tot0)
                init.extend([zeros16] * (2 * len(kvecs)))
            out = lax.fori_loop(0, nchunks, sbody, tuple(init), unroll=2)
            res = []
            ci = 0
            for _, _, kvecs in specs:
                ci += 1
                cur = []
                for _ in kvecs:
                    cur.append((jnp.sum(out[ci]), jnp.max(out[ci + 1])))
                    ci += 2
                res.append(cur)
            return res

        begin_round(_U, _R1_BINS)

        @plsc.parallel_loop(0, per_tile // L, unroll=_U)
        def _(i):
            v = nrm_v[pl.ds(i * L, L)]
            bits = plsc.bitcast(v, jnp.int32)
            hi = bits >> 20
            u = (i % _U) * 4096
            plsc.addupdate_scatter(hist_v, [zeros16, u + hi], ones16)
        merge_hist(_U, _R1_BINS)

        half = _R1_BINS // 2

        def pbody(i, acc):
            return acc + mhist_v[0, pl.ds(i * L, L)]
        accv = lax.fori_loop(0, half // L, pbody, zeros16, unroll=8)
        htvec = jnp.full((L,), jnp.sum(accv))

        r1 = scan_multi([(0, zeros16, [ka_vec, kb_vec]),
                         (half, htvec, [ka_vec, kb_vec])], half // L)
        b1a = r1[0][0][0] + r1[1][0][0]
        c1a = jnp.maximum(r1[0][0][1], r1[1][0][1])
        b1b = r1[0][1][0] + r1[1][1][0]
        c1b = jnp.maximum(r1[0][1][1], r1[1][1][1])
        ka = jnp.max(ka_vec)
        kb = jnp.max(kb_vec)
        r2a_vec = jnp.full((L,), ka - c1a)
        r2b_vec = jnp.full((L,), kb - c1b)

        begin_round(_U, 2 * _R2_BINS)

        @plsc.parallel_loop(0, per_tile // L, unroll=_U)
        def _(i):
            v = nrm_v[pl.ds(i * L, L)]
            bits = plsc.bitcast(v, jnp.int32)
            hi = bits >> 20
            mid = (bits >> 9) & 0x7FF
            u = (i % _U) * 4096
            plsc.addupdate_scatter(hist_v, [zeros16, u + mid],
                                   ones16, mask=hi == b1a)
            plsc.addupdate_scatter(hist_v, [zeros16, u + _R2_BINS + mid],
                                   ones16, mask=hi == b1b)
        merge_hist(_U, 2 * _R2_BINS)

        r2 = scan_multi([(0, zeros16, [r2a_vec]),
                         (_R2_BINS, zeros16, [r2b_vec])], _R2_BINS // L)
        b2a, c2a = r2[0][0]
        b2b, c2b = r2[1][0]
        r3a_vec = r2a_vec - c2a
        r3b_vec = r2b_vec - c2b

        begin_round(_U, 2 * _R2_BINS)

        @plsc.parallel_loop(0, per_tile // L, unroll=_U)
        def _(i):
            v = nrm_v[pl.ds(i * L, L)]
            bits = plsc.bitcast(v, jnp.int32)
            hi = bits >> 20
            mid = (bits >> 9) & 0x7FF
            lo = bits & 0x1FF
            u = (i % _U) * 4096
            plsc.addupdate_scatter(hist_v, [zeros16, u + lo], ones16,
                                   mask=(hi == b1a) & (mid == b2a))
            plsc.addupdate_scatter(hist_v, [zeros16, u + _R2_BINS + lo],
                                   ones16, mask=(hi == b1b) & (mid == b2b))
        merge_hist(_U, 2 * _R2_BINS)

        r3 = scan_multi([(0, zeros16, [r3a_vec]),
                         (_R2_BINS, zeros16, [r3b_vec])], _R3_BINS // L)
        b3a = r3[0][0][0]
        b3b = r3[1][0][0]

        bits_a = (b1a << 20) | (b2a << 9) | b3a
        bits_b = (b1b << 20) | (b2b << 9) | b3b
        va = plsc.bitcast(jnp.full((L,), bits_a, jnp.int32), jnp.float32)
        vb = plsc.bitcast(jnp.full((L,), bits_b, jnp.int32), jnp.float32)
        res_v[...] = va + frac_vec * (vb - va)

        @pl.when(sid == 0)
        def _():
            pltpu.sync_copy(res_v, out_hbm.at[cid])

    return qkernel


def _make_p3_body(scale):
    def _p3_body(nr_ref, se_ref, qv_ref, pq_ref, w_ref, out_ref):
        i = pl.program_id(0)
        n = nr_ref[...]
        se = se_ref[...]
        q_lo = qv_ref[0, 0]
        q_hi = qv_ref[1, 0]
        tw = jnp.where(n < q_lo, w_ref[0], 0.0)
        tw = jnp.where((n >= pq_ref[1]) & (n < pq_ref[2]), w_ref[1], tw)
        tw = jnp.where(n > q_hi, w_ref[2], tw)
        part = jnp.sum(tw * se).reshape(1, 1)

        @pl.when(i == 0)
        def _():
            out_ref[...] = jnp.zeros((1, 1), jnp.float32)

        out_ref[...] += part

        @pl.when(i == pl.num_programs(0) - 1)
        def _():
            out_ref[...] = out_ref[...] * scale
    return _p3_body


def _pass3(norms, sqerr, qv, pq, w, total, blk=65536):
    n = norms.shape[0]
    smem = pl.BlockSpec(memory_space=pltpu.SMEM)
    return pl.pallas_call(
        _make_p3_body(1.0 / total),
        grid=(n // blk,),
        in_specs=[
            pl.BlockSpec((blk,), lambda i: (i,)),
            pl.BlockSpec((blk,), lambda i: (i,)),
            smem, smem, smem,
        ],
        out_specs=pl.BlockSpec((1, 1), lambda i: (0, 0)),
        out_shape=jax.ShapeDtypeStruct((1, 1), jnp.float32),
    )(norms, sqerr, qv, pq, w)


def kernel(input, target, quantiles, weights):
    n, d = target.shape
    sqerr, norms = _pass1(input, target)

    zero1 = jnp.zeros((1,), jnp.int32)
    qv = _make_quantile_kernel(n, int(quantiles.shape[0]))(
        norms, quantiles.astype(jnp.float32), zero1)
    loss = _pass3(norms, sqerr, qv, quantiles.astype(jnp.float32),
                  weights.astype(jnp.float32), float(n) * float(d))
    return loss.reshape(())
